# baseline (speedup 1.0000x reference)
import functools

import jax
import jax.numpy as jnp
from jax import lax
from jax.experimental import pallas as pl
from jax.experimental.pallas import tpu as pltpu
from jax.experimental.pallas import tpu_sc as plsc

_ROWS = 8192
_COLS = 768
_NUM_WORKERS = 32
_ROWS_PER_WORKER = _ROWS // _NUM_WORKERS
_CHUNK_ROWS = 128
_CHUNKS = _ROWS_PER_WORKER // _CHUNK_ROWS

_mesh = plsc.VectorSubcoreMesh(core_axis_name="c", subcore_axis_name="s")


@functools.partial(
    pl.kernel,
    mesh=_mesh,
    out_type=jax.ShapeDtypeStruct((_ROWS, _COLS), jnp.float32),
    scratch_types=[pltpu.VMEM((_CHUNK_ROWS, _COLS), jnp.float32)],
)
def _sc_copy(x_hbm, o_hbm, buf):
    wid = lax.axis_index("s") * 2 + lax.axis_index("c")
    base = wid * _ROWS_PER_WORKER
    for c in range(_CHUNKS):
        start = base + c * _CHUNK_ROWS
        pltpu.sync_copy(x_hbm.at[pl.ds(start, _CHUNK_ROWS), :], buf)
        pltpu.sync_copy(buf, o_hbm.at[pl.ds(start, _CHUNK_ROWS), :])


def kernel(prototypes):
    return _sc_copy(prototypes)
